# baseline (speedup 1.0000x reference)
import functools

import jax
import jax.numpy as jnp
from jax.experimental import pallas as pl
from jax.experimental.pallas import tpu as pltpu

B, D, H, O, E = 2048, 1024, 2048, 1024, 8


def _gate_body(x_ref, gw_ref, gb_ref, out_ref, acc_ref):
    e = pl.program_id(0)
    x = x_ref[0]
    gw = gw_ref[0]
    part = jax.lax.dot_general(x, gw, (((1,), (1,)), ((), ())),
                               preferred_element_type=jnp.float32)

    @pl.when(e == 0)
    def _():
        acc_ref[...] = part + gb_ref[...][None, :]

    @pl.when(e > 0)
    def _():
        acc_ref[...] += part

    @pl.when(e == E - 1)
    def _():
        l = acc_ref[...]
        cols = jax.lax.broadcasted_iota(jnp.int32, (B, E), 1)
        a1 = jnp.argmax(l, axis=1)[:, None]
        m1 = jnp.max(l, axis=1)[:, None]
        l2 = jnp.where(cols == a1, -jnp.inf, l)
        a2 = jnp.argmax(l2, axis=1)[:, None]
        m2 = jnp.max(l2, axis=1)[:, None]
        z = jnp.exp(m2 - m1)
        w1 = 1.0 / (1.0 + z)
        w2 = 1.0 - w1
        out_ref[...] = jnp.where(cols == a1, w1,
                                 jnp.where(cols == a2, w2, 0.0))


def _expert_body(x_ref, wi_ref, bi_ref, wo_ref, bo_ref, dw_ref, out_ref, h_ref):
    e = pl.program_id(0)
    x = x_ref[0]
    h = jax.lax.dot_general(x, wi_ref[0], (((1,), (1,)), ((), ())),
                            preferred_element_type=jnp.float32)
    h = jnp.maximum(h + bi_ref[0], 0.0)
    h_ref[...] = h.astype(jnp.bfloat16)
    o = jax.lax.dot_general(h_ref[...], wo_ref[0], (((1,), (1,)), ((), ())),
                            preferred_element_type=jnp.float32)
    o = o + bo_ref[0]
    onehot = (jax.lax.broadcasted_iota(jnp.int32, (E, 1), 0) == e
              ).astype(jnp.float32)
    col = jnp.dot(dw_ref[...], onehot,
                  preferred_element_type=jnp.float32)
    contrib = o * col

    @pl.when(e == 0)
    def _():
        out_ref[...] = contrib

    @pl.when(e > 0)
    def _():
        out_ref[...] += contrib


@jax.jit
def kernel(trait_0, trait_1, trait_2, trait_3, trait_4, trait_5, trait_6,
           trait_7, gate_W, gate_b, W_in, b_in, W_out, b_out):
    traits = [trait_0, trait_1, trait_2, trait_3, trait_4, trait_5,
              trait_6, trait_7]
    X = jnp.stack(traits, axis=0)
    gWr = gate_W.reshape(E, E, D).transpose(1, 0, 2)

    dense_w = pl.pallas_call(
        _gate_body,
        grid=(E,),
        in_specs=[
            pl.BlockSpec((1, B, D), lambda e: (e, 0, 0)),
            pl.BlockSpec((1, E, D), lambda e: (e, 0, 0)),
            pl.BlockSpec((E,), lambda e: (0,)),
        ],
        out_specs=pl.BlockSpec((B, E), lambda e: (0, 0)),
        out_shape=jax.ShapeDtypeStruct((B, E), jnp.float32),
        scratch_shapes=[pltpu.VMEM((B, E), jnp.float32)],
    )(X, gWr, gate_b)

    out = pl.pallas_call(
        _expert_body,
        grid=(E,),
        in_specs=[
            pl.BlockSpec((1, B, D), lambda e: (e, 0, 0)),
            pl.BlockSpec((1, H, D), lambda e: (e, 0, 0)),
            pl.BlockSpec((1, 1, H), lambda e: (e, 0, 0)),
            pl.BlockSpec((1, O, H), lambda e: (e, 0, 0)),
            pl.BlockSpec((1, 1, O), lambda e: (e, 0, 0)),
            pl.BlockSpec((B, E), lambda e: (0, 0)),
        ],
        out_specs=pl.BlockSpec((B, O), lambda e: (0, 0)),
        out_shape=jax.ShapeDtypeStruct((B, O), jnp.float32),
        scratch_shapes=[pltpu.VMEM((B, H), jnp.bfloat16)],
    )(X.astype(jnp.bfloat16), W_in.astype(jnp.bfloat16),
      b_in.reshape(E, 1, H), W_out.astype(jnp.bfloat16),
      b_out.reshape(E, 1, O), dense_w)
    return out
